# Initial kernel scaffold; baseline (speedup 1.0000x reference)
import jax
import jax.numpy as jnp
from jax.experimental import pallas as pl


def kernel(x, edge_index, etypes, W, W_loop, b):
    raise NotImplementedError("write your pallas kernel here")


# speedup vs baseline: 3.1347x; 3.1347x over previous
import functools

import jax
import jax.numpy as jnp
from jax import lax
from jax.experimental import pallas as pl
from jax.experimental.pallas import tpu as pltpu
from jax.experimental.pallas import tpu_sc as plsc

NC = 2
NS = 16
NW = NC * NS


def _transform_body(x_ref, wt_ref, wl_ref, t_ref, lp_ref):
    xb = x_ref[...]
    t_ref[...] = jnp.dot(xb, wt_ref[...], preferred_element_type=jnp.float32)
    lp_ref[...] = jnp.dot(xb, wl_ref[...], preferred_element_type=jnp.float32)


def _transform(x, wt, wl, n_blk):
    n, d = x.shape
    rdo = wt.shape[1]
    do = wl.shape[1]
    grid = n // n_blk
    return pl.pallas_call(
        _transform_body,
        grid=(grid,),
        in_specs=[
            pl.BlockSpec((n_blk, d), lambda i: (i, 0)),
            pl.BlockSpec((d, rdo), lambda i: (0, 0)),
            pl.BlockSpec((d, do), lambda i: (0, 0)),
        ],
        out_specs=[
            pl.BlockSpec((n_blk, rdo), lambda i: (i, 0)),
            pl.BlockSpec((n_blk, do), lambda i: (i, 0)),
        ],
        out_shape=[
            jax.ShapeDtypeStruct((n, rdo), jnp.float32),
            jax.ShapeDtypeStruct((n, do), jnp.float32),
        ],
    )(x, wt, wl)


def _make_sc_agg(n_nodes, n_edges, do, r):
    MC = 2000
    GC = 80
    NSUB = MC // GC
    epw = n_edges // NW
    nmeta = epw // MC
    ZR = 80
    RPT = 640
    assert epw % MC == 0 and MC % GC == 0 and GC % 16 == 0
    assert (NS - 1) * RPT < n_nodes <= NS * RPT
    assert (n_nodes - (NS - 1) * RPT) % ZR == 0 and RPT % ZR == 0

    mesh = plsc.VectorSubcoreMesh(core_axis_name="c", subcore_axis_name="s")

    @functools.partial(
        pl.kernel,
        out_type=jax.ShapeDtypeStruct((NC, n_nodes, do), jnp.float32),
        mesh=mesh,
        scratch_types=[
            pltpu.VMEM((MC,), jnp.int32),
            pltpu.VMEM((MC,), jnp.int32),
            pltpu.VMEM((MC,), jnp.int32),
            pltpu.VMEM((NSUB, GC), jnp.int32),
            pltpu.VMEM((GC,), jnp.int32),
            pltpu.VMEM((GC, do), jnp.float32),
            pltpu.VMEM((ZR, do), jnp.float32),
            pltpu.VMEM_SHARED((n_nodes, do), jnp.float32),
            pltpu.SemaphoreType.DMA,
        ],
    )
    def sc_agg(t_hbm, src_hbm, dstm_hbm, et_hbm, out_hbm,
               src_v, dstm_v, et_v, dst_v, gidx_v, rows_v, zbuf_v, acc_sh,
               sem):
        cid = lax.axis_index("c")
        sid = lax.axis_index("s")
        wid = sid * NC + cid

        zero16 = jnp.zeros((16,), jnp.float32)

        def zrow(i, carry):
            for j in range(do // 16):
                zbuf_v[i, pl.ds(j * 16, 16)] = zero16
            return carry

        lax.fori_loop(0, ZR, zrow, 0)
        row0 = sid * RPT
        ncopies = jnp.where(sid < NS - 1, RPT // ZR,
                            (n_nodes - (NS - 1) * RPT) // ZR)

        def zcopy(k, carry):
            pltpu.sync_copy(
                zbuf_v, acc_sh.at[pl.ds(pl.multiple_of(row0 + k * ZR, ZR), ZR)])
            return carry

        lax.fori_loop(0, ncopies, zcopy, 0)
        plsc.subcore_barrier()

        ebase = wid * epw

        def meta_round(m, carry):
            base = pl.multiple_of(ebase + m * MC, MC)
            pltpu.sync_copy(src_hbm.at[pl.ds(base, MC)], src_v)
            pltpu.sync_copy(et_hbm.at[pl.ds(base, MC)], et_v)
            pltpu.sync_copy(dstm_hbm.at[pl.ds(base, MC)], dstm_v)

            def sub(g, carry3):
                goff = pl.multiple_of(g * GC, GC)
                for j in range(GC // 16):
                    s = src_v[pl.ds(goff + j * 16, 16)]
                    t = et_v[pl.ds(goff + j * 16, 16)]
                    gidx_v[pl.ds(j * 16, 16)] = s * r + t
                    dst_v[g, pl.ds(j * 16, 16)] = dstm_v[pl.ds(goff + j * 16, 16)]
                pltpu.async_copy(t_hbm.at[gidx_v], rows_v, sem).wait()
                pltpu.sync_copy(rows_v, acc_sh.at[dst_v.at[g]], add=True)
                return carry3

            lax.fori_loop(0, NSUB, sub, 0)
            return carry

        lax.fori_loop(0, nmeta, meta_round, 0)
        plsc.subcore_barrier()

        def wcopy(k, carry):
            off = pl.multiple_of(row0 + k * ZR, ZR)
            pltpu.sync_copy(acc_sh.at[pl.ds(off, ZR)],
                            out_hbm.at[cid, pl.ds(off, ZR)])
            return carry

        lax.fori_loop(0, ncopies, wcopy, 0)

    return sc_agg


def _combine_body(p_ref, lp_ref, b_ref, o_ref):
    p = p_ref[...]
    o_ref[...] = p[0] + p[1] + lp_ref[...] + b_ref[...]


def _combine(partials, loop_out, b, n_blk):
    _, n, do = partials.shape
    grid = n // n_blk
    return pl.pallas_call(
        _combine_body,
        grid=(grid,),
        in_specs=[
            pl.BlockSpec((NC, n_blk, do), lambda i: (0, i, 0)),
            pl.BlockSpec((n_blk, do), lambda i: (i, 0)),
            pl.BlockSpec((1, do), lambda i: (0, 0)),
        ],
        out_specs=pl.BlockSpec((n_blk, do), lambda i: (i, 0)),
        out_shape=jax.ShapeDtypeStruct((n, do), jnp.float32),
    )(partials, loop_out, b.reshape(1, do))


def kernel(x, edge_index, etypes, W, W_loop, b):
    n, d = x.shape
    r, _, do = W.shape
    e = etypes.shape[0]

    wt = jnp.transpose(W, (1, 0, 2)).reshape(d, r * do)

    transformed, loop_out = _transform(x, wt, W_loop, n_blk=1000)
    t_rows = transformed.reshape(n * r, do)

    src = edge_index[0].astype(jnp.int32)
    dst = edge_index[1].astype(jnp.int32)
    et = etypes.astype(jnp.int32)

    partials = _make_sc_agg(n, e, do, r)(t_rows, src, dst, et)
    return _combine(partials, loop_out, b, n_blk=1000)


# speedup vs baseline: 4.1555x; 1.3257x over previous
import functools

import jax
import jax.numpy as jnp
from jax import lax
from jax.experimental import pallas as pl
from jax.experimental.pallas import tpu as pltpu
from jax.experimental.pallas import tpu_sc as plsc

NC = 2
NS = 16
NW = NC * NS


def _transform_body(x_ref, wt_ref, wl_ref, t_ref, lp_ref):
    xb = x_ref[...]
    t_ref[...] = jnp.dot(xb, wt_ref[...], preferred_element_type=jnp.float32)
    lp_ref[...] = jnp.dot(xb, wl_ref[...], preferred_element_type=jnp.float32)


def _transform(x, wt, wl, n_blk):
    n, d = x.shape
    rdo = wt.shape[1]
    do = wl.shape[1]
    grid = n // n_blk
    return pl.pallas_call(
        _transform_body,
        grid=(grid,),
        in_specs=[
            pl.BlockSpec((n_blk, d), lambda i: (i, 0)),
            pl.BlockSpec((d, rdo), lambda i: (0, 0)),
            pl.BlockSpec((d, do), lambda i: (0, 0)),
        ],
        out_specs=[
            pl.BlockSpec((n_blk, rdo), lambda i: (i, 0)),
            pl.BlockSpec((n_blk, do), lambda i: (i, 0)),
        ],
        out_shape=[
            jax.ShapeDtypeStruct((n, rdo), jnp.float32),
            jax.ShapeDtypeStruct((n, do), jnp.float32),
        ],
    )(x, wt, wl)


def _make_sc_agg(n_nodes, n_edges, do, r):
    MC = 2000
    GC = 80
    NSUB = MC // GC
    epw = n_edges // NW
    nmeta = epw // MC
    ZR = 80
    RPT = 640
    assert epw % MC == 0 and MC % GC == 0 and GC % 16 == 0
    assert (NS - 1) * RPT < n_nodes <= NS * RPT
    assert (n_nodes - (NS - 1) * RPT) % ZR == 0 and RPT % ZR == 0

    mesh = plsc.VectorSubcoreMesh(core_axis_name="c", subcore_axis_name="s")

    @functools.partial(
        pl.kernel,
        out_type=jax.ShapeDtypeStruct((NC, n_nodes, do), jnp.float32),
        mesh=mesh,
        scratch_types=[
            pltpu.VMEM((MC,), jnp.int32),
            pltpu.VMEM((MC,), jnp.int32),
            pltpu.VMEM((MC,), jnp.int32),
            pltpu.VMEM((NSUB, GC), jnp.int32),
            pltpu.VMEM((GC,), jnp.int32),
            pltpu.VMEM((GC,), jnp.int32),
            pltpu.VMEM((GC, do), jnp.float32),
            pltpu.VMEM((GC, do), jnp.float32),
            pltpu.VMEM((ZR, do), jnp.float32),
            pltpu.VMEM_SHARED((n_nodes, do), jnp.float32),
            pltpu.SemaphoreType.DMA,
            pltpu.SemaphoreType.DMA,
        ],
    )
    def sc_agg(t_hbm, src_hbm, dstm_hbm, et_hbm, out_hbm,
               src_v, dstm_v, et_v, dst_v, gidx0_v, gidx1_v, rows0_v,
               rows1_v, zbuf_v, acc_sh, sem0, sem1):
        cid = lax.axis_index("c")
        sid = lax.axis_index("s")
        wid = sid * NC + cid

        zero16 = jnp.zeros((16,), jnp.float32)

        def zrow(i, carry):
            for j in range(do // 16):
                zbuf_v[i, pl.ds(j * 16, 16)] = zero16
            return carry

        lax.fori_loop(0, ZR, zrow, 0)
        row0 = sid * RPT
        ncopies = jnp.where(sid < NS - 1, RPT // ZR,
                            (n_nodes - (NS - 1) * RPT) // ZR)

        def zcopy(k, carry):
            pltpu.sync_copy(
                zbuf_v, acc_sh.at[pl.ds(pl.multiple_of(row0 + k * ZR, ZR), ZR)])
            return carry

        lax.fori_loop(0, ncopies, zcopy, 0)
        plsc.subcore_barrier()

        ebase = wid * epw
        nchunks = epw // GC

        def load_meta(mr):
            base = pl.multiple_of(ebase + mr * MC, MC)
            pltpu.sync_copy(src_hbm.at[pl.ds(base, MC)], src_v)
            pltpu.sync_copy(et_hbm.at[pl.ds(base, MC)], et_v)
            pltpu.sync_copy(dstm_hbm.at[pl.ds(base, MC)], dstm_v)

        def prep_fire(c, gidx_b, rows_b, sem_b):
            @pl.when(c % NSUB == 0)
            def _():
                load_meta(c // NSUB)

            rr = c % NSUB
            goff = rr * GC
            for j in range(GC // 16):
                s = src_v[pl.ds(goff + j * 16, 16)]
                t = et_v[pl.ds(goff + j * 16, 16)]
                gidx_b[pl.ds(j * 16, 16)] = s * r + t
                dst_v[rr, pl.ds(j * 16, 16)] = dstm_v[pl.ds(goff + j * 16, 16)]
            pltpu.async_copy(t_hbm.at[gidx_b], rows_b, sem_b)

        def drain(c, gidx_b, rows_b, sem_b):
            pltpu.make_async_copy(t_hbm.at[gidx_b], rows_b, sem_b).wait()
            pltpu.sync_copy(rows_b, acc_sh.at[dst_v.at[c % NSUB]], add=True)

        prep_fire(0, gidx0_v, rows0_v, sem0)

        def pair(h, carry):
            c = 2 * h
            prep_fire(c + 1, gidx1_v, rows1_v, sem1)
            drain(c, gidx0_v, rows0_v, sem0)
            prep_fire(c + 2, gidx0_v, rows0_v, sem0)
            drain(c + 1, gidx1_v, rows1_v, sem1)
            return carry

        lax.fori_loop(0, (nchunks - 1) // 2, pair, 0)
        drain(nchunks - 1, gidx0_v, rows0_v, sem0)
        plsc.subcore_barrier()

        def wcopy(k, carry):
            off = pl.multiple_of(row0 + k * ZR, ZR)
            pltpu.sync_copy(acc_sh.at[pl.ds(off, ZR)],
                            out_hbm.at[cid, pl.ds(off, ZR)])
            return carry

        lax.fori_loop(0, ncopies, wcopy, 0)

    return sc_agg


def _combine_body(p_ref, lp_ref, b_ref, o_ref):
    p = p_ref[...]
    o_ref[...] = p[0] + p[1] + lp_ref[...] + b_ref[...]


def _combine(partials, loop_out, b, n_blk):
    _, n, do = partials.shape
    grid = n // n_blk
    return pl.pallas_call(
        _combine_body,
        grid=(grid,),
        in_specs=[
            pl.BlockSpec((NC, n_blk, do), lambda i: (0, i, 0)),
            pl.BlockSpec((n_blk, do), lambda i: (i, 0)),
            pl.BlockSpec((1, do), lambda i: (0, 0)),
        ],
        out_specs=pl.BlockSpec((n_blk, do), lambda i: (i, 0)),
        out_shape=jax.ShapeDtypeStruct((n, do), jnp.float32),
    )(partials, loop_out, b.reshape(1, do))


def kernel(x, edge_index, etypes, W, W_loop, b):
    n, d = x.shape
    r, _, do = W.shape
    e = etypes.shape[0]

    wt = jnp.transpose(W, (1, 0, 2)).reshape(d, r * do)

    transformed, loop_out = _transform(x, wt, W_loop, n_blk=1000)
    t_rows = transformed.reshape(n * r, do)

    src = edge_index[0].astype(jnp.int32)
    dst = edge_index[1].astype(jnp.int32)
    et = etypes.astype(jnp.int32)

    partials = _make_sc_agg(n, e, do, r)(t_rows, src, dst, et)
    return _combine(partials, loop_out, b, n_blk=1000)
